# Initial kernel scaffold; baseline (speedup 1.0000x reference)
import jax
import jax.numpy as jnp
from jax.experimental import pallas as pl


def kernel(feat, segment_ids, W1, b1, W2, b2, W3, b3):
    raise NotImplementedError("write your pallas kernel here")


# speedup vs baseline: 5.2853x; 5.2853x over previous
import jax
import jax.numpy as jnp
from jax.experimental import pallas as pl
from jax.experimental.pallas import tpu as pltpu

N_NODES = 100000
D_FEAT = 128
NUM_GRAPHS = 512
HIDDEN = 256
BLK = 2000
NBLK = N_NODES // BLK


def _body(seg_ref, feat_ref, W1_ref, b1_ref, W2_ref, b2_ref, W3_ref, b3_ref,
          out_ref, acc_ref, cnt_ref):
    i = pl.program_id(0)

    @pl.when(i == 0)
    def _init():
        acc_ref[...] = jnp.zeros_like(acc_ref)
        cnt_ref[...] = jnp.zeros_like(cnt_ref)

    segs = seg_ref[0, 0, :]
    oh = (segs[:, None] == jax.lax.broadcasted_iota(
        jnp.int32, (BLK, NUM_GRAPHS), 1)).astype(jnp.float32)
    acc_ref[...] += jax.lax.dot_general(
        oh, feat_ref[...], (((0,), (0,)), ((), ())),
        preferred_element_type=jnp.float32)
    cnt_ref[...] += jax.lax.dot_general(
        oh, jnp.ones((BLK, 8), jnp.float32), (((0,), (0,)), ((), ())),
        preferred_element_type=jnp.float32)

    @pl.when(i == NBLK - 1)
    def _finish():
        counts = jnp.maximum(cnt_ref[:, 0:1], 1.0)
        pooled = acc_ref[...] / counts
        h = jnp.maximum(
            jnp.dot(pooled, W1_ref[...], preferred_element_type=jnp.float32)
            + b1_ref[...], 0.0)
        h = jnp.maximum(
            jnp.dot(h, W2_ref[...], preferred_element_type=jnp.float32)
            + b2_ref[...], 0.0)
        out_ref[...] = (
            jnp.dot(h, W3_ref[...], preferred_element_type=jnp.float32)
            + b3_ref[...])


def kernel(feat, segment_ids, W1, b1, W2, b2, W3, b3):
    seg3 = segment_ids.astype(jnp.int32).reshape(NBLK, 1, BLK)
    pred = pl.pallas_call(
        _body,
        grid=(NBLK,),
        in_specs=[
            pl.BlockSpec((1, 1, BLK), lambda i: (i, 0, 0)),
            pl.BlockSpec((BLK, D_FEAT), lambda i: (i, 0)),
            pl.BlockSpec((D_FEAT, HIDDEN), lambda i: (0, 0)),
            pl.BlockSpec((1, HIDDEN), lambda i: (0, 0)),
            pl.BlockSpec((HIDDEN, HIDDEN), lambda i: (0, 0)),
            pl.BlockSpec((1, HIDDEN), lambda i: (0, 0)),
            pl.BlockSpec((HIDDEN, 1), lambda i: (0, 0)),
            pl.BlockSpec((1, 1), lambda i: (0, 0)),
        ],
        out_specs=pl.BlockSpec((NUM_GRAPHS, 1), lambda i: (0, 0)),
        out_shape=jax.ShapeDtypeStruct((NUM_GRAPHS, 1), jnp.float32),
        scratch_shapes=[
            pltpu.VMEM((NUM_GRAPHS, D_FEAT), jnp.float32),
            pltpu.VMEM((NUM_GRAPHS, 8), jnp.float32),
        ],
    )(seg3, feat, W1, b1.reshape(1, HIDDEN), W2, b2.reshape(1, HIDDEN),
      W3, b3.reshape(1, 1))
    return pred.reshape(NUM_GRAPHS)
